# Initial kernel scaffold; baseline (speedup 1.0000x reference)
import jax
import jax.numpy as jnp
from jax.experimental import pallas as pl


def kernel(x_0, x_1, x_2, laplacian_0, laplacian_down_1, laplacian_up_1, laplacian_down_2, laplacian_up_2, incidence_1, incidence_2, w_0, w_1, w_2):
    raise NotImplementedError("write your pallas kernel here")


# speedup vs baseline: 1.4727x; 1.4727x over previous
import functools

import jax
import jax.numpy as jnp
from jax.experimental import pallas as pl

F32 = jnp.float32
BF16 = jnp.bfloat16
C = 128


def _mm_nn_body(a_ref, b_ref, o_ref, *, relu):
    a = a_ref[...].astype(BF16)
    b = b_ref[...].astype(BF16)
    acc = jax.lax.dot_general(a, b, (((1,), (0,)), ((), ())),
                              preferred_element_type=F32)
    if relu:
        acc = jnp.maximum(acc, 0.0)
    o_ref[...] = acc.astype(o_ref.dtype)


def _mm_nn(a, b, bm, out_dtype=BF16, relu=False):
    M, K = a.shape
    K2, N = b.shape
    assert K == K2 and M % bm == 0, (a.shape, b.shape, bm)
    return pl.pallas_call(
        functools.partial(_mm_nn_body, relu=relu),
        grid=(M // bm,),
        in_specs=[pl.BlockSpec((bm, K), lambda i: (i, 0)),
                  pl.BlockSpec((K, N), lambda i: (0, 0))],
        out_specs=pl.BlockSpec((bm, N), lambda i: (i, 0)),
        out_shape=jax.ShapeDtypeStruct((M, N), out_dtype),
    )(a, b)


def _mm_tn_body(a_ref, b_ref, o_ref):
    a = a_ref[...].astype(BF16)
    b = b_ref[...].astype(BF16)
    acc = jax.lax.dot_general(a, b, (((0,), (0,)), ((), ())),
                              preferred_element_type=F32)

    @pl.when(pl.program_id(0) == 0)
    def _():
        o_ref[...] = jnp.zeros_like(o_ref)

    o_ref[...] += acc


def _mm_tn(a, b, bk):
    K, M = a.shape
    K2, N = b.shape
    assert K == K2 and K % bk == 0, (a.shape, b.shape, bk)
    return pl.pallas_call(
        _mm_tn_body,
        grid=(K // bk,),
        in_specs=[pl.BlockSpec((bk, M), lambda i: (i, 0)),
                  pl.BlockSpec((bk, N), lambda i: (i, 0))],
        out_specs=pl.BlockSpec((M, N), lambda i: (0, 0)),
        out_shape=jax.ShapeDtypeStruct((M, N), F32),
    )(a, b)


def kernel(x_0, x_1, x_2, laplacian_0, laplacian_down_1, laplacian_up_1,
           laplacian_down_2, laplacian_up_2, incidence_1, incidence_2,
           w_0, w_1, w_2):
    t0 = _mm_nn(incidence_1, x_1, 400)
    p01 = _mm_tn(incidence_1, x_0, 400).astype(BF16)
    t21 = _mm_nn(incidence_2, x_2, 400)
    p12 = _mm_tn(incidence_2, x_1, 400).astype(BF16)

    x0b = x_0.astype(BF16)
    x1b = x_1.astype(BF16)
    x2b = x_2.astype(BF16)

    B0 = jnp.concatenate([x0b, t0], axis=1)
    c1 = _mm_nn(laplacian_0, B0, 400)
    c2 = _mm_nn(laplacian_0, c1, 400)

    B1 = jnp.concatenate([x1b, p01, t21], axis=1)
    d1 = _mm_nn(laplacian_down_1, B1, 400)
    d2 = _mm_nn(laplacian_down_1, d1, 400)
    u1 = _mm_nn(laplacian_up_1, B1, 400)
    u2 = _mm_nn(laplacian_up_1, u1, 400)

    B2 = jnp.concatenate([x2b, p12], axis=1)
    e1 = _mm_nn(laplacian_down_2, B2, 400)
    e2 = _mm_nn(laplacian_down_2, e1, 400)
    f1 = _mm_nn(laplacian_up_2, B2, 400)
    f2 = _mm_nn(laplacian_up_2, f1, 400)

    s = lambda a, k: a[:, k * C:(k + 1) * C]
    MSG0 = jnp.concatenate(
        [x0b, s(c1, 0), s(c2, 0), t0, s(c1, 1), s(c2, 1)], axis=1)
    MSG1 = jnp.concatenate(
        [x1b, s(d1, 0), s(d2, 0), s(u1, 0), s(u2, 0),
         p01, s(d1, 1), s(d2, 1), s(u1, 1), s(u2, 1),
         t21, s(d1, 2), s(d2, 2), s(u1, 2), s(u2, 2)], axis=1)
    MSG2 = jnp.concatenate(
        [x2b, s(e1, 0), s(e2, 0), s(f1, 0), s(f2, 0),
         p12, s(e1, 1), s(e2, 1), s(f1, 1), s(f2, 1)], axis=1)

    W0 = jnp.transpose(w_0, (2, 0, 1)).reshape(6 * C, C)
    W1 = jnp.transpose(w_1, (2, 0, 1)).reshape(15 * C, C)
    W2 = jnp.transpose(w_2, (2, 0, 1)).reshape(10 * C, C)

    y0 = _mm_nn(MSG0, W0, 400, out_dtype=F32, relu=True)
    y1 = _mm_nn(MSG1, W1, 400, out_dtype=F32, relu=True)
    y2 = _mm_nn(MSG2, W2, 400, out_dtype=F32, relu=True)
    return (y0, y1, y2)


# speedup vs baseline: 1.8689x; 1.2691x over previous
import functools

import jax
import jax.numpy as jnp
from jax.experimental import pallas as pl

F32 = jnp.float32
BF16 = jnp.bfloat16
C = 128


def _bdot(a, b):
    return jax.lax.dot_general(a.astype(BF16), b.astype(BF16),
                               (((1,), (0,)), ((), ())),
                               preferred_element_type=F32)


def _bdot_tn(a, b):
    return jax.lax.dot_general(a.astype(BF16), b.astype(BF16),
                               (((0,), (0,)), ((), ())),
                               preferred_element_type=F32)


def _inc_body(a_ref, xa_ref, xb_ref, o1_ref, o2_ref):
    a = a_ref[...]
    o1_ref[...] = _bdot(a, xa_ref[...]).astype(o1_ref.dtype)

    @pl.when(pl.program_id(0) == 0)
    def _():
        o2_ref[...] = jnp.zeros_like(o2_ref)

    o2_ref[...] += _bdot_tn(a, xb_ref[...])


def _incidence(a, xa, xb, bm):
    M, N = a.shape
    return pl.pallas_call(
        _inc_body,
        grid=(M // bm,),
        in_specs=[pl.BlockSpec((bm, N), lambda i: (i, 0)),
                  pl.BlockSpec((N, C), lambda i: (0, 0)),
                  pl.BlockSpec((bm, C), lambda i: (i, 0))],
        out_specs=[pl.BlockSpec((bm, C), lambda i: (i, 0)),
                   pl.BlockSpec((N, C), lambda i: (0, 0))],
        out_shape=[jax.ShapeDtypeStruct((M, C), BF16),
                   jax.ShapeDtypeStruct((N, C), F32)],
    )(a, xa, xb)


def _cheb1_body(l_ref, b_ref, o_ref):
    o_ref[...] = _bdot(l_ref[...], b_ref[...]).astype(o_ref.dtype)


def _cheb1(l, b, bm):
    M = l.shape[0]
    F = b.shape[1]
    return pl.pallas_call(
        _cheb1_body,
        grid=(M // bm,),
        in_specs=[pl.BlockSpec((bm, M), lambda i: (i, 0)),
                  pl.BlockSpec((M, F), lambda i: (0, 0))],
        out_specs=pl.BlockSpec((bm, F), lambda i: (i, 0)),
        out_shape=jax.ShapeDtypeStruct((M, F), BF16),
    )(l, b)


def _cheb2_body(la_ref, lb_ref, ba_ref, bb_ref, oa_ref, ob_ref):
    oa_ref[...] = _bdot(la_ref[...], ba_ref[...]).astype(oa_ref.dtype)
    ob_ref[...] = _bdot(lb_ref[...], bb_ref[...]).astype(ob_ref.dtype)


def _cheb2(la, lb, ba, bb, bm):
    M = la.shape[0]
    F = ba.shape[1]
    return pl.pallas_call(
        _cheb2_body,
        grid=(M // bm,),
        in_specs=[pl.BlockSpec((bm, M), lambda i: (i, 0)),
                  pl.BlockSpec((bm, M), lambda i: (i, 0)),
                  pl.BlockSpec((M, F), lambda i: (0, 0)),
                  pl.BlockSpec((M, F), lambda i: (0, 0))],
        out_specs=[pl.BlockSpec((bm, F), lambda i: (i, 0)),
                   pl.BlockSpec((bm, F), lambda i: (i, 0))],
        out_shape=[jax.ShapeDtypeStruct((M, F), BF16),
                   jax.ShapeDtypeStruct((M, F), BF16)],
    )(la, lb, ba, bb)


def _head_body(*refs, n_in):
    in_refs = refs[:n_in]
    w_refs = refs[n_in:2 * n_in]
    o_ref = refs[2 * n_in]
    acc = _bdot(in_refs[0][...], w_refs[0][...])
    for x_ref, w_ref in zip(in_refs[1:], w_refs[1:]):
        acc += _bdot(x_ref[...], w_ref[...])
    o_ref[...] = jnp.maximum(acc, 0.0)


def _head(inputs, weights, bm):
    n = len(inputs)
    M = inputs[0].shape[0]
    in_specs = [pl.BlockSpec((bm, x.shape[1]), lambda i: (i, 0))
                for x in inputs]
    in_specs += [pl.BlockSpec(w.shape, lambda i: (0, 0)) for w in weights]
    return pl.pallas_call(
        functools.partial(_head_body, n_in=n),
        grid=(M // bm,),
        in_specs=in_specs,
        out_specs=pl.BlockSpec((bm, C), lambda i: (i, 0)),
        out_shape=jax.ShapeDtypeStruct((M, C), F32),
    )(*inputs, *weights)


def kernel(x_0, x_1, x_2, laplacian_0, laplacian_down_1, laplacian_up_1,
           laplacian_down_2, laplacian_up_2, incidence_1, incidence_2,
           w_0, w_1, w_2):
    x0b = x_0.astype(BF16)
    x1b = x_1.astype(BF16)
    x2b = x_2.astype(BF16)

    t0, p01 = _incidence(incidence_1, x_1, x_0, 400)
    t21, p12 = _incidence(incidence_2, x_2, x_1, 400)
    p01 = p01.astype(BF16)
    p12 = p12.astype(BF16)

    B0 = jnp.concatenate([x0b, t0], axis=1)
    c1 = _cheb1(laplacian_0, B0, 400)
    c2 = _cheb1(laplacian_0, c1, 400)

    B1 = jnp.concatenate([x1b, p01, t21], axis=1)
    d1, u1 = _cheb2(laplacian_down_1, laplacian_up_1, B1, B1, 400)
    d2, u2 = _cheb2(laplacian_down_1, laplacian_up_1, d1, u1, 400)

    B2 = jnp.concatenate([x2b, p12], axis=1)
    e1, f1 = _cheb2(laplacian_down_2, laplacian_up_2, B2, B2, 400)
    e2, f2 = _cheb2(laplacian_down_2, laplacian_up_2, e1, f1, 400)

    wt0 = jnp.transpose(w_0, (2, 0, 1)).astype(BF16)
    wt1 = jnp.transpose(w_1, (2, 0, 1)).astype(BF16)
    wt2 = jnp.transpose(w_2, (2, 0, 1)).astype(BF16)
    g = lambda wt, idx: wt[jnp.array(idx)].reshape(len(idx) * C, C)

    y0 = _head([B0, c1, c2],
               [g(wt0, [0, 3]), g(wt0, [1, 4]), g(wt0, [2, 5])], 400)
    y1 = _head([B1, d1, d2, u1, u2],
               [g(wt1, [0, 5, 10]), g(wt1, [1, 6, 11]), g(wt1, [2, 7, 12]),
                g(wt1, [3, 8, 13]), g(wt1, [4, 9, 14])], 400)
    y2 = _head([B2, e1, e2, f1, f2],
               [g(wt2, [0, 5]), g(wt2, [1, 6]), g(wt2, [2, 7]),
                g(wt2, [3, 8]), g(wt2, [4, 9])], 400)
    return (y0, y1, y2)
